# baseline (speedup 1.0000x reference)
import functools

import jax
import jax.numpy as jnp
from jax import lax
from jax.experimental import pallas as pl
from jax.experimental.pallas import tpu as pltpu
from jax.experimental.pallas import tpu_sc as plsc

D_MODEL = 64
_SCALE = 8.0

_NC = 2
_NS = 16
_NW = _NC * _NS

_SBLK = 128
_NBUF = 4
_DEPTH = 3


@functools.partial(jax.jit, static_argnames=("n_cols", "n_seq"))
def _sc_embed(xt, table, *, n_cols, n_seq):
    mesh = plsc.VectorSubcoreMesh(
        core_axis_name="c", subcore_axis_name="s",
        num_cores=_NC, num_subcores=_NS,
    )

    @functools.partial(
        pl.kernel,
        out_type=jax.ShapeDtypeStruct((n_cols, D_MODEL, n_seq), jnp.float32),
        mesh=mesh,
        compiler_params=pltpu.CompilerParams(
            use_tc_tiling_on_sc=False, needs_layout_passes=False),
        scratch_types=[
            pltpu.VMEM((n_cols, _SBLK), jnp.int32),
            [pltpu.VMEM((_SBLK, D_MODEL), jnp.float32) for _ in range(_NBUF)],
            [pltpu.VMEM((D_MODEL, _SBLK), jnp.float32) for _ in range(_NBUF)],
            [pltpu.SemaphoreType.DMA for _ in range(_NBUF)],
            [pltpu.SemaphoreType.DMA for _ in range(_NBUF)],
        ],
    )
    def k(xt_hbm, tab_hbm, out_hbm, idx_all, gbuf, tbuf, gsem, ssem):
        wid = lax.axis_index("s") * _NC + lax.axis_index("c")
        s0 = wid * _SBLK

        pltpu.sync_copy(xt_hbm.at[:, pl.ds(s0, _SBLK)], idx_all)

        def start_gather(c, b):
            pltpu.async_copy(tab_hbm.at[idx_all.at[c]], gbuf[b], gsem[b])

        def wait_gather(c, b):
            pltpu.make_async_copy(
                tab_hbm.at[idx_all.at[c]], gbuf[b], gsem[b]).wait()

        def start_scatter(c, b):
            pltpu.async_copy(
                tbuf[b], out_hbm.at[c, :, pl.ds(s0, _SBLK)], ssem[b])

        def wait_scatter(b):
            pltpu.make_async_copy(
                tbuf[b], out_hbm.at[0, :, pl.ds(s0, _SBLK)], ssem[b]).wait()

        row_ids = [
            lax.iota(jnp.int32, 16) + jnp.int32(16 * g)
            for g in range(_SBLK // 16)
        ]

        def transpose_scale(b):
            def per_feature(d, carry):
                col = jnp.full((16,), d, dtype=jnp.int32)
                for g in range(_SBLK // 16):
                    v = plsc.load_gather(gbuf[b], [row_ids[g], col])
                    tbuf[b][d, pl.ds(16 * g, 16)] = v * _SCALE
                return carry
            lax.fori_loop(0, D_MODEL, per_feature, 0)

        for c0 in range(_DEPTH):
            start_gather(c0, c0)

        def outer(i, carry):
            for b in range(_NBUF):
                c = i * _NBUF + b
                wait_gather(c, b)

                cn = c + _DEPTH
                bn = (b + _DEPTH) % _NBUF

                @pl.when(cn < n_cols)
                def _prefetch():
                    start_gather(cn, bn)

                @pl.when(c >= _NBUF)
                def _drain():
                    wait_scatter(b)

                transpose_scale(b)
                start_scatter(c, b)
            return carry

        lax.fori_loop(0, n_cols // _NBUF, outer, 0)

        for b in range(_NBUF):
            wait_scatter(b)

    return k(xt, table)


def kernel(x, table):
    S, C = x.shape
    xt = x.T.astype(jnp.int32)
    out_t = _sc_embed(xt, table, n_cols=C, n_seq=S)
    return out_t.transpose(2, 0, 1)


# speedup vs baseline: 1.0731x; 1.0731x over previous
import functools

import jax
import jax.numpy as jnp
from jax import lax
from jax.experimental import pallas as pl
from jax.experimental.pallas import tpu as pltpu
from jax.experimental.pallas import tpu_sc as plsc

D_MODEL = 64
_SCALE = 8.0

_NC = 2
_NS = 16
_NW = _NC * _NS

_SBLK = 128
_GPAD = 129
_NBUF = 4
_IBUF = 8
_DEPTH = 3


@functools.partial(jax.jit, static_argnames=("n_cols", "n_seq"))
def _sc_embed(xt, tab2, *, n_cols, n_seq):
    mesh = plsc.VectorSubcoreMesh(
        core_axis_name="c", subcore_axis_name="s",
        num_cores=_NC, num_subcores=_NS,
    )
    n_st = n_seq // _SBLK

    @functools.partial(
        pl.kernel,
        out_type=jax.ShapeDtypeStruct(
            (n_cols, D_MODEL // 8, n_st, 8, _SBLK), jnp.float32),
        mesh=mesh,
        compiler_params=pltpu.CompilerParams(
            use_tc_tiling_on_sc=False, needs_layout_passes=False),
        scratch_types=[
            [pltpu.VMEM((_SBLK,), jnp.int32) for _ in range(_IBUF)],
            [pltpu.VMEM((_SBLK,), jnp.int32) for _ in range(_NBUF)],
            [pltpu.VMEM((_SBLK, 128), jnp.float32) for _ in range(_NBUF)],
            pltpu.VMEM((_SBLK, _GPAD), jnp.float32),
            [pltpu.VMEM((D_MODEL // 8, 8, _SBLK), jnp.float32)
             for _ in range(_NBUF)],
            [pltpu.SemaphoreType.DMA for _ in range(_IBUF)],
            [pltpu.SemaphoreType.DMA for _ in range(_NBUF)],
            [pltpu.SemaphoreType.DMA for _ in range(_NBUF)],
        ],
    )
    def k(xt_hbm, tab_hbm, out_hbm, ibuf, gidx, gbuf, gpad, tbuf,
          isem, gsem, ssem):
        wid = lax.axis_index("s") * _NC + lax.axis_index("c")
        s0 = wid * _SBLK

        def start_idx(c, j):
            pltpu.async_copy(
                xt_hbm.at[c, pl.ds(s0, _SBLK)], ibuf[j], isem[j])

        def start_gather(c, b, j):
            pltpu.make_async_copy(
                xt_hbm.at[0, pl.ds(s0, _SBLK)], ibuf[j], isem[j]).wait()
            for g in range(_SBLK // 16):
                sl = pl.ds(16 * g, 16)
                gidx[b][sl] = lax.shift_right_logical(ibuf[j][sl], 1)
            pltpu.async_copy(tab_hbm.at[gidx[b]], gbuf[b], gsem[b])

        def wait_gather(b):
            pltpu.make_async_copy(
                tab_hbm.at[gidx[b]], gbuf[b], gsem[b]).wait()

        def start_scatter(c, b):
            pltpu.async_copy(tbuf[b], out_hbm.at[c, :, wid], ssem[b])

        def wait_scatter(b):
            pltpu.make_async_copy(
                tbuf[b], out_hbm.at[0, :, wid], ssem[b]).wait()

        row_ids = [
            lax.iota(jnp.int32, 16) + jnp.int32(16 * g)
            for g in range(_SBLK // 16)
        ]

        def repack(b):
            def rows(r, carry):
                for rr in range(2):
                    for g in range(128 // 16):
                        sl = pl.ds(16 * g, 16)
                        gpad[r * 2 + rr, sl] = gbuf[b][r * 2 + rr, sl]
                return carry
            lax.fori_loop(0, _SBLK // 2, rows, 0)

        def transpose_scale(b, j):
            tb = tbuf[b]
            for g in range(_SBLK // 16):
                sl = pl.ds(16 * g, 16)
                half = lax.shift_left(
                    lax.bitwise_and(ibuf[j][sl], jnp.int32(1)),
                    jnp.int32(6))

                def per_dt(dt, carry):
                    col0 = half + dt * 8
                    for di in range(8):
                        v = plsc.load_gather(gpad, [row_ids[g], col0 + di])
                        tb[dt, di, sl] = v * _SCALE
                    return carry
                lax.fori_loop(0, D_MODEL // 8, per_dt, 0)

        for j0 in range(_DEPTH + 1):
            start_idx(j0, j0)
        for c0 in range(_DEPTH):
            start_gather(c0, c0, c0)

        def outer(i, carry):
            for b8 in range(_IBUF):
                c = i * _IBUF + b8
                b = b8 % _NBUF

                ci = c + _DEPTH + 1
                @pl.when(ci < n_cols)
                def _previdx():
                    start_idx(ci, (b8 + _DEPTH + 1) % _IBUF)

                cn = c + _DEPTH
                bn = (b + _DEPTH) % _NBUF
                @pl.when(cn < n_cols)
                def _prefetch():
                    start_gather(cn, bn, (b8 + _DEPTH) % _IBUF)

                wait_gather(b)
                repack(b)

                @pl.when(c >= _NBUF)
                def _drain():
                    wait_scatter(b)

                transpose_scale(b, b8)
                start_scatter(c, b)
            return carry

        lax.fori_loop(0, n_cols // _IBUF, outer, 0)

        for b in range(_NBUF):
            wait_scatter(b)

    return k(xt, tab2)


def kernel(x, table):
    S, C = x.shape
    xt = x.T.astype(jnp.int32)
    tab2 = table.reshape(-1, 128)
    out5 = _sc_embed(xt, tab2, n_cols=C, n_seq=S)
    return (out5.transpose(2, 4, 0, 1, 3)
            .reshape(S, C, D_MODEL))


# speedup vs baseline: 1.7400x; 1.6215x over previous
import functools

import jax
import jax.numpy as jnp
from jax import lax
from jax.experimental import pallas as pl
from jax.experimental.pallas import tpu as pltpu
from jax.experimental.pallas import tpu_sc as plsc

D_MODEL = 64
_SCALE = 8.0

_NC = 2
_NS = 16
_NW = _NC * _NS

_CHUNK = 128
_NBUF = 8
_DEPTH = 3
_ROWS_PER_ITER = 4


@functools.partial(jax.jit, static_argnames=("b_per_w", "n_chunks"))
def _sc_embed(x3, table, *, b_per_w, n_chunks):
    B = _NW * b_per_w
    mesh = plsc.VectorSubcoreMesh(
        core_axis_name="c", subcore_axis_name="s",
        num_cores=_NC, num_subcores=_NS,
    )

    @functools.partial(
        pl.kernel,
        out_type=jax.ShapeDtypeStruct((B, D_MODEL), jnp.float32),
        mesh=mesh,
        compiler_params=pltpu.CompilerParams(use_tc_tiling_on_sc=False),
        scratch_types=[
            pltpu.VMEM((n_chunks, _CHUNK), jnp.int32),
            [pltpu.VMEM((_CHUNK, D_MODEL), jnp.float32) for _ in range(_NBUF)],
            [pltpu.SemaphoreType.DMA for _ in range(_NBUF)],
            [pltpu.SemaphoreType.DMA for _ in range(_NBUF)],
        ],
    )
    def k(x_hbm, tab_hbm, out_hbm, idx_all, rows, gsem, ssem):
        wid = lax.axis_index("s") * _NC + lax.axis_index("c")
        base = wid * b_per_w

        pltpu.sync_copy(x_hbm.at[wid], idx_all)

        def start_gather(g, b):
            pltpu.async_copy(tab_hbm.at[idx_all.at[g]], rows[b], gsem[b])

        def wait_gather(g, b):
            pltpu.make_async_copy(
                tab_hbm.at[idx_all.at[g]], rows[b], gsem[b]).wait()

        def start_scatter(g, b):
            pltpu.async_copy(
                rows[b], out_hbm.at[pl.ds(base + g * _CHUNK, _CHUNK)], ssem[b])

        def wait_scatter(b):
            pltpu.make_async_copy(
                rows[b], out_hbm.at[pl.ds(base, _CHUNK)], ssem[b]).wait()

        def scale(b):
            def srows(i, carry):
                for dr in range(_ROWS_PER_ITER):
                    r = i * _ROWS_PER_ITER + dr
                    for c in range(D_MODEL // 16):
                        sl = pl.ds(16 * c, 16)
                        rows[b][r, sl] = rows[b][r, sl] * _SCALE
                return carry
            lax.fori_loop(0, _CHUNK // _ROWS_PER_ITER, srows, 0)

        for g0 in range(_DEPTH):
            start_gather(g0, g0)

        def outer(i, carry):
            for b in range(_NBUF):
                g = i * _NBUF + b
                wait_gather(g, b)
                scale(b)
                start_scatter(g, b)
                gn = g + _DEPTH
                bn = (b + _DEPTH) % _NBUF

                @pl.when(gn < n_chunks)
                def _prefetch():
                    @pl.when(gn >= _NBUF)
                    def _drain():
                        wait_scatter(bn)
                    start_gather(gn, bn)
            return carry

        lax.fori_loop(0, n_chunks // _NBUF, outer, 0)

        for b in range(_NBUF):
            wait_scatter(b)

    return k(x3, table)


def kernel(x, table):
    S, L = x.shape
    B = S * L
    b_per_w = B // _NW
    n_chunks = b_per_w // _CHUNK
    x3 = x.reshape(_NW, n_chunks, _CHUNK).astype(jnp.int32)
    out = _sc_embed(x3, table, b_per_w=b_per_w, n_chunks=n_chunks)
    return out.reshape(S, L, D_MODEL)
